# baseline (speedup 1.0000x reference)
import jax
import jax.numpy as jnp
from jax import lax
from jax.experimental import pallas as pl
from jax.experimental.pallas import tpu as pltpu
from jax.experimental.pallas import tpu_sc as plsc

VOCAB = 1000000
EMBED_DIM = 64
ROW_PAD = 128
NUM_PROMPT = 3
PID_BASE = VOCAB - NUM_PROMPT
TAIL = 64
TAIL_BASE = VOCAB - TAIL

NC, NS, L = 2, 16, 16
NW = NC * NS
PH0 = 128
PH1 = 72
TBLK = 512


def _tr_body(tabt_hbm, tpad_hbm, src_v, dst_v):
    n_rows = tabt_hbm.shape[1]
    n_blk = n_rows // TBLK
    wid = lax.axis_index("s") * NC + lax.axis_index("c")

    iota16 = lax.iota(jnp.int32, L)

    def do_blk(b, _):
        r0 = pl.multiple_of(b * TBLK, TBLK)
        pltpu.sync_copy(tabt_hbm.at[:, pl.ds(r0, TBLK)], src_v)

        def tr_row(c, _):
            csplat = jnp.broadcast_to(c, (L,)).astype(jnp.int32)
            for q in range(EMBED_DIM // L):
                vals = plsc.load_gather(src_v, [iota16 + q * L, csplat])
                dst_v[c, pl.ds(q * L, L)] = vals
            return 0

        lax.fori_loop(0, TBLK, tr_row, 0)
        pltpu.sync_copy(dst_v, tpad_hbm.at[pl.ds(r0, TBLK)])
        return 0

    lax.fori_loop(0, n_blk // NW, lambda i, _: do_blk(i * NW + wid, 0), 0)
    last = (n_blk // NW) * NW + wid

    @pl.when(last < n_blk)
    def _():
        do_blk(last, 0)


def _sc_body(idxt_hbm, table_hbm, tail_hbm, pp_hbm, out_hbm,
             idxblk_v, idxb0_v, idxb1_v, rows_v,
             patch_v, pp_v, sem0, sem1):
    seq_len, n_seq = idxt_hbm.shape
    seq_per_w = n_seq // NW

    wid = lax.axis_index("s") * NC + lax.axis_index("c")
    base_seq = wid * seq_per_w

    pltpu.sync_copy(pp_hbm, pp_v)
    pltpu.sync_copy(tail_hbm, patch_v)
    pltpu.sync_copy(
        idxt_hbm.at[:, pl.ds(pl.multiple_of(base_seq, 128), seq_per_w)],
        idxblk_v)
    for r in range(NUM_PROMPT):
        for q in range(EMBED_DIM // L):
            patch_v[TAIL - NUM_PROMPT + r, pl.ds(q * L, L)] = \
                pp_v[r, pl.ds(q * L, L)]

    iota16 = lax.iota(jnp.int32, L)

    def do_seq(g, _):
        scol = jnp.broadcast_to(g, (L,)).astype(jnp.int32)

        def fill0(j, _):
            p_vec = j * L + iota16
            vals = plsc.load_gather(idxblk_v, [p_vec, scol])
            idxb0_v[pl.ds(j * L, L)] = jnp.clip(vals, 0, VOCAB - 1)
            return 0

        lax.fori_loop(0, PH0 // L, fill0, 0)

        def fill1(j, _):
            p_vec = jnp.minimum(PH0 + j * L + iota16, seq_len - 1)
            vals = plsc.load_gather(idxblk_v, [p_vec, scol])
            idxb1_v[pl.ds(j * L, L)] = jnp.clip(vals, 0, VOCAB - 1)
            return 0

        lax.fori_loop(0, (PH1 + L - 1) // L, fill1, 0)

        pltpu.async_copy(table_hbm.at[idxb0_v],
                         rows_v.at[pl.ds(0, PH0)], sem0)
        pltpu.async_copy(table_hbm.at[idxb1_v.at[pl.ds(0, PH1)]],
                         rows_v.at[pl.ds(PH0, PH1)], sem1)

        def fix_win(base, off):
            ivec = idxb0_v[pl.ds(off, L)] if base == 0 else \
                idxb1_v[pl.ds(off, L)]
            cond = ivec >= TAIL_BASE
            cnt = jnp.sum(cond.astype(jnp.int32))

            @pl.when(cnt > 0)
            def _():
                def fix_lane(_, mask):
                    lane = jnp.min(jnp.where(mask > 0, iota16, L))
                    kk = jnp.max(
                        jnp.where(iota16 == lane, ivec - TAIL_BASE, -1))
                    row_splat = jnp.broadcast_to(
                        base + off + lane, (L,)).astype(jnp.int32)
                    for q in range(EMBED_DIM // L):
                        val = patch_v[kk, pl.ds(q * L, L)]
                        plsc.store_scatter(
                            rows_v, [row_splat, iota16 + q * L], val)
                    return mask & (iota16 != lane).astype(jnp.int32)

                lax.fori_loop(0, cnt, fix_lane, cond.astype(jnp.int32))

        seq = base_seq + g
        pltpu.make_async_copy(table_hbm.at[idxb0_v],
                              rows_v.at[pl.ds(0, PH0)], sem0).wait()
        for w in range(PH0 // L):
            fix_win(0, w * L)
        pltpu.make_async_copy(table_hbm.at[idxb1_v.at[pl.ds(0, PH1)]],
                              rows_v.at[pl.ds(PH0, PH1)], sem1).wait()
        for w in range((PH1 + L - 1) // L):
            fix_win(PH0, min(w * L, PH1 - L))
        pltpu.sync_copy(rows_v, out_hbm.at[seq])
        return 0

    lax.fori_loop(0, seq_per_w, do_seq, 0)


@jax.jit
def _run(idxt, tabt, tail, pp):
    seq_len, n_seq = idxt.shape
    n_rows = tabt.shape[1]
    mesh = plsc.VectorSubcoreMesh(core_axis_name="c", subcore_axis_name="s")
    cp = pltpu.CompilerParams(
        use_tc_tiling_on_sc=True, needs_layout_passes=False)

    tpad = pl.kernel(
        _tr_body,
        out_type=jax.ShapeDtypeStruct((n_rows, ROW_PAD), jnp.float32),
        mesh=mesh,
        scratch_types=[
            pltpu.VMEM((EMBED_DIM, TBLK), jnp.float32),
            pltpu.VMEM((TBLK, ROW_PAD), jnp.float32),
        ],
        compiler_params=cp,
    )(tabt)

    return pl.kernel(
        _sc_body,
        out_type=jax.ShapeDtypeStruct((n_seq, seq_len, ROW_PAD), jnp.float32),
        mesh=mesh,
        scratch_types=[
            pltpu.VMEM((200, 128), jnp.int32),
            pltpu.VMEM((PH0,), jnp.int32),
            pltpu.VMEM((PH0,), jnp.int32),
            pltpu.VMEM((200, ROW_PAD), jnp.float32),
            pltpu.VMEM((TAIL, EMBED_DIM), jnp.float32),
            pltpu.VMEM((NUM_PROMPT, EMBED_DIM), jnp.float32),
            pltpu.SemaphoreType.DMA,
            pltpu.SemaphoreType.DMA,
        ],
        compiler_params=cp,
    )(idxt, tpad, tail, pp)


def kernel(input, table, prompt_params):
    out = _run(input.astype(jnp.int32).T, table.T,
               table[VOCAB - TAIL:], prompt_params)
    return out[:, :, :EMBED_DIM]
